# baseline (speedup 1.0000x reference)
import functools

import jax
import jax.numpy as jnp
from jax import lax
from jax.experimental import pallas as pl
from jax.experimental.pallas import tpu as pltpu
from jax.experimental.pallas import tpu_sc as plsc

_B, _N, _M = 4, 8192, 2048
_C1, _C2 = 64, 128
_COUT = 128
_TN = 512
_NB = _N // _TN
_TN2 = 1024
_NB2 = _N // _TN2

_HI = jax.lax.Precision.HIGHEST

_NW = 32
_PPW = _B * _N // _NW
_CH = 128
_NCH = _PPW // _CH


def _nn3w(kc_ref, uc_ref, w_ref, i_ref):
    b = pl.program_id(0)

    kc = kc_ref[0]
    uc = uc_ref[0]
    inner = jnp.dot(kc.astype(jnp.bfloat16), uc.astype(jnp.bfloat16),
                    preferred_element_type=jnp.float32)
    k2 = kc[:, 0:1] * kc[:, 0:1] + kc[:, 1:2] * kc[:, 1:2] + kc[:, 2:3] * kc[:, 2:3]
    u2 = uc[0:1, :] * uc[0:1, :] + uc[1:2, :] * uc[1:2, :] + uc[2:3, :] * uc[2:3, :]
    d2 = jnp.maximum(u2 + k2 - 2.0 * inner, 0.0)

    inf = jnp.float32(jnp.inf)
    iota = jax.lax.broadcasted_iota(jnp.int32, (_M, _TN), 0)
    big = jnp.int32(_M)

    m1 = jnp.min(d2, axis=0, keepdims=True)
    i1 = jnp.min(jnp.where(d2 == m1, iota, big), axis=0, keepdims=True)
    d2a = jnp.where(iota == i1, inf, d2)
    m2 = jnp.min(d2a, axis=0, keepdims=True)
    i2 = jnp.min(jnp.where(d2a == m2, iota, big), axis=0, keepdims=True)
    d2b = jnp.where(iota == i2, inf, d2a)
    m3 = jnp.min(d2b, axis=0, keepdims=True)
    i3 = jnp.min(jnp.where(d2b == m3, iota, big), axis=0, keepdims=True)

    r1 = 1.0 / (m1 + 1e-8)
    r2 = 1.0 / (m2 + 1e-8)
    r3 = 1.0 / (m3 + 1e-8)
    norm = r1 + r2 + r3

    w_ref[0, 0:1, :] = r1 / norm
    w_ref[0, 1:2, :] = r2 / norm
    w_ref[0, 2:3, :] = r3 / norm
    w_ref[0, 3:8, :] = jnp.zeros((5, _TN), jnp.float32)
    off = b * _M
    i_ref[0, 0:1, :] = i1 + off
    i_ref[0, 1:2, :] = i2 + off
    i_ref[0, 2:3, :] = i3 + off
    i_ref[0, 3:8, :] = jnp.zeros((5, _TN), jnp.int32)


def _gt(kf_ref, w0_ref, gt_ref):
    kf = kf_ref[0].astype(jnp.bfloat16)
    w0a = w0_ref[:, :_C2].astype(jnp.bfloat16)
    gt_ref[...] = jax.lax.dot_general(
        kf, w0a, (((0,), (1,)), ((), ())),
        preferred_element_type=jnp.float32)


def _interp_sc(gt_hbm, idx_hbm, wgt_hbm, y_hbm, idx_v, w_v, rows_v, y_v, sem):
    wid = lax.axis_index("s") * 2 + lax.axis_index("c")
    for ch in range(_NCH):
        base = wid * _PPW + ch * _CH
        pltpu.sync_copy(wgt_hbm.at[pl.ds(base * 3, _CH * 3)],
                        w_v.at[pl.ds(0, _CH * 3)])
        for j in range(3):
            pltpu.sync_copy(idx_hbm.at[pl.ds(j * _B * _N + base, _CH)],
                            idx_v.at[pl.ds(j * _CH, _CH)])
            pltpu.async_copy(
                gt_hbm.at[idx_v.at[pl.ds(j * _CH, _CH)]],
                rows_v.at[pl.ds(j * _CH, _CH)], sem).wait()

        def body(p, _):
            wv = w_v[pl.ds(3 * p, 16)]
            w0 = wv[0]
            w1 = wv[1]
            w2 = wv[2]
            for c in range(_COUT // 16):
                sl = pl.ds(c * 16, 16)
                y_v[p, sl] = (rows_v[p, sl] * w0
                              + rows_v[_CH + p, sl] * w1
                              + rows_v[2 * _CH + p, sl] * w2)
            return _

        lax.fori_loop(0, _CH, body, 0)
        pltpu.sync_copy(y_v, y_hbm.at[pl.ds(base, _CH)])


def _mlp_from_y(y_ref, uf_ref, w0_ref, x_ref, sums_ref):
    b = pl.program_id(0)
    nb = pl.program_id(1)

    @pl.when(jnp.logical_and(b == 0, nb == 0))
    def _():
        sums_ref[...] = jnp.zeros_like(sums_ref)

    x = jnp.swapaxes(y_ref[0], 0, 1)
    x = x + jnp.dot(w0_ref[:, _C2:].astype(jnp.bfloat16),
                    uf_ref[0].astype(jnp.bfloat16),
                    preferred_element_type=jnp.float32)
    x_ref[0] = x
    sums_ref[:, 0:1] += jnp.sum(x, axis=1, keepdims=True)
    sums_ref[:, 1:2] += jnp.sum(x * x, axis=1, keepdims=True)


def _fp_norm(x_ref, sums_ref, gm_ref, bt_ref, o_ref):
    cnt = jnp.float32(_B * _N)
    mean = sums_ref[:, 0:1] / cnt
    var = sums_ref[:, 1:2] / cnt - mean * mean
    inv = jax.lax.rsqrt(var + 1e-5)
    scale = gm_ref[...] * inv
    shift = bt_ref[...] - mean * scale
    o_ref[0] = jnp.maximum(x_ref[0] * scale + shift, 0.0)


def kernel(unknown, known, unknow_feats, known_feats, W0, gamma0, beta0):
    uc = jnp.concatenate(
        [jnp.swapaxes(unknown, 1, 2),
         jnp.zeros((_B, 5, _N), jnp.float32)], axis=1)
    kc = jnp.concatenate(
        [known, jnp.zeros((_B, _M, 5), jnp.float32)], axis=2)

    wgt, idx = pl.pallas_call(
        _nn3w,
        grid=(_B, _NB),
        in_specs=[
            pl.BlockSpec((1, _M, 8), lambda b, n: (b, 0, 0)),
            pl.BlockSpec((1, 8, _TN), lambda b, n: (b, 0, n)),
        ],
        out_specs=[
            pl.BlockSpec((1, 8, _TN), lambda b, n: (b, 0, n)),
            pl.BlockSpec((1, 8, _TN), lambda b, n: (b, 0, n)),
        ],
        out_shape=[
            jax.ShapeDtypeStruct((_B, 8, _N), jnp.float32),
            jax.ShapeDtypeStruct((_B, 8, _N), jnp.int32),
        ],
        compiler_params=pltpu.CompilerParams(
            dimension_semantics=("arbitrary", "arbitrary")),
    )(kc, uc)

    gt = pl.pallas_call(
        _gt,
        grid=(_B,),
        in_specs=[
            pl.BlockSpec((1, _C2, _M), lambda b: (b, 0, 0)),
            pl.BlockSpec((_COUT, _C1 + _C2), lambda b: (0, 0)),
        ],
        out_specs=pl.BlockSpec((_M, _COUT), lambda b: (b, 0)),
        out_shape=jax.ShapeDtypeStruct((_B * _M, _COUT), jnp.float32),
    )(known_feats, W0)

    idx_flat = jnp.swapaxes(idx[:, :3, :], 0, 1).reshape(3 * _B * _N)
    wgt_flat = jnp.swapaxes(wgt[:, :3, :], 1, 2).reshape(_B * _N * 3)

    sc_interp = functools.partial(
        pl.kernel,
        out_type=jax.ShapeDtypeStruct((_B * _N, _COUT), jnp.float32),
        mesh=plsc.VectorSubcoreMesh(core_axis_name="c", subcore_axis_name="s"),
        scratch_types=[
            pltpu.VMEM((_CH * 3,), jnp.int32),
            pltpu.VMEM((_CH * 3 + 16,), jnp.float32),
            pltpu.VMEM((_CH * 3, _COUT), jnp.float32),
            pltpu.VMEM((_CH, _COUT), jnp.float32),
            pltpu.SemaphoreType.DMA,
        ],
    )(_interp_sc)

    y = sc_interp(gt, idx_flat, wgt_flat)
    y = y.reshape(_B, _N, _COUT)

    x_pre, sums = pl.pallas_call(
        _mlp_from_y,
        grid=(_B, _NB),
        in_specs=[
            pl.BlockSpec((1, _TN, _COUT), lambda b, n: (b, n, 0)),
            pl.BlockSpec((1, _C1, _TN), lambda b, n: (b, 0, n)),
            pl.BlockSpec((_COUT, _C1 + _C2), lambda b, n: (0, 0)),
        ],
        out_specs=[
            pl.BlockSpec((1, _COUT, _TN), lambda b, n: (b, 0, n)),
            pl.BlockSpec((_COUT, 8), lambda b, n: (0, 0)),
        ],
        out_shape=[
            jax.ShapeDtypeStruct((_B, _COUT, _N), jnp.float32),
            jax.ShapeDtypeStruct((_COUT, 8), jnp.float32),
        ],
        compiler_params=pltpu.CompilerParams(
            dimension_semantics=("arbitrary", "arbitrary")),
    )(y, unknow_feats, W0)

    out = pl.pallas_call(
        _fp_norm,
        grid=(_B, _NB2),
        in_specs=[
            pl.BlockSpec((1, _COUT, _TN2), lambda b, n: (b, 0, n)),
            pl.BlockSpec((_COUT, 8), lambda b, n: (0, 0)),
            pl.BlockSpec((_COUT, 1), lambda b, n: (0, 0)),
            pl.BlockSpec((_COUT, 1), lambda b, n: (0, 0)),
        ],
        out_specs=pl.BlockSpec((1, _COUT, _TN2), lambda b, n: (b, 0, n)),
        out_shape=jax.ShapeDtypeStruct((_B, _COUT, _N), jnp.float32),
        compiler_params=pltpu.CompilerParams(
            dimension_semantics=("arbitrary", "arbitrary")),
    )(x_pre, sums, gamma0.reshape(_COUT, 1), beta0.reshape(_COUT, 1))

    return out


# speedup vs baseline: 1.0687x; 1.0687x over previous
import functools

import jax
import jax.numpy as jnp
from jax import lax
from jax.experimental import pallas as pl
from jax.experimental.pallas import tpu as pltpu
from jax.experimental.pallas import tpu_sc as plsc

_B, _N, _M = 4, 8192, 2048
_C1, _C2 = 64, 128
_COUT = 128
_TN = 512
_NB = _N // _TN
_TN2 = 1024
_NB2 = _N // _TN2

_HI = jax.lax.Precision.HIGHEST

_NW = 32
_PPW = _B * _N // _NW
_CH = 128
_NCH = _PPW // _CH


def _nn3w(kc_ref, uc_ref, kf_ref, w0_ref, w_ref, i_ref, gt_ref):
    b = pl.program_id(0)
    nb = pl.program_id(1)

    @pl.when(nb == 0)
    def _():
        kf = kf_ref[0].astype(jnp.bfloat16)
        w0a = w0_ref[:, :_C2].astype(jnp.bfloat16)
        gt_ref[...] = jax.lax.dot_general(
            kf, w0a, (((0,), (1,)), ((), ())),
            preferred_element_type=jnp.float32)

    kc = kc_ref[0]
    uc = uc_ref[0]
    inner = jnp.dot(kc.astype(jnp.bfloat16), uc.astype(jnp.bfloat16),
                    preferred_element_type=jnp.float32)
    k2 = kc[:, 0:1] * kc[:, 0:1] + kc[:, 1:2] * kc[:, 1:2] + kc[:, 2:3] * kc[:, 2:3]
    u2 = uc[0:1, :] * uc[0:1, :] + uc[1:2, :] * uc[1:2, :] + uc[2:3, :] * uc[2:3, :]
    d2 = jnp.maximum(u2 + k2 - 2.0 * inner, 0.0)

    inf = jnp.float32(jnp.inf)
    iota = jax.lax.broadcasted_iota(jnp.int32, (_M, _TN), 0)
    big = jnp.int32(_M)

    m1 = jnp.min(d2, axis=0, keepdims=True)
    i1 = jnp.min(jnp.where(d2 == m1, iota, big), axis=0, keepdims=True)
    d2a = jnp.where(iota == i1, inf, d2)
    m2 = jnp.min(d2a, axis=0, keepdims=True)
    i2 = jnp.min(jnp.where(d2a == m2, iota, big), axis=0, keepdims=True)
    d2b = jnp.where(iota == i2, inf, d2a)
    m3 = jnp.min(d2b, axis=0, keepdims=True)
    i3 = jnp.min(jnp.where(d2b == m3, iota, big), axis=0, keepdims=True)

    r1 = 1.0 / (m1 + 1e-8)
    r2 = 1.0 / (m2 + 1e-8)
    r3 = 1.0 / (m3 + 1e-8)
    norm = r1 + r2 + r3

    w_ref[0, 0:1, :] = r1 / norm
    w_ref[0, 1:2, :] = r2 / norm
    w_ref[0, 2:3, :] = r3 / norm
    w_ref[0, 3:8, :] = jnp.zeros((5, _TN), jnp.float32)
    off = b * _M
    i_ref[0, 0:1, :] = i1 + off
    i_ref[0, 1:2, :] = i2 + off
    i_ref[0, 2:3, :] = i3 + off
    i_ref[0, 3:8, :] = jnp.zeros((5, _TN), jnp.int32)


def _interp_sc(gt_hbm, idx_hbm, wgt_hbm, y_hbm, idx_v, w_v0, w_v1, rows_v,
               y_v, sem0, sem1):
    wid = lax.axis_index("s") * 2 + lax.axis_index("c")
    sems = (sem0, sem1)
    wvs = (w_v0, w_v1)

    def fire(ch):
        par = ch % 2
        base = wid * _PPW + ch * _CH
        pltpu.sync_copy(wgt_hbm.at[pl.ds(base * 3, _CH * 3)],
                        wvs[par].at[pl.ds(0, _CH * 3)])
        for j in range(3):
            pltpu.sync_copy(idx_hbm.at[pl.ds(j * _B * _N + base, _CH)],
                            idx_v.at[par, pl.ds(j * _CH, _CH)])
            pltpu.async_copy(
                gt_hbm.at[idx_v.at[par, pl.ds(j * _CH, _CH)]],
                rows_v.at[par, pl.ds(j * _CH, _CH)], sems[par])

    def drain(ch):
        par = ch % 2
        for j in range(3):
            pltpu.make_async_copy(
                gt_hbm.at[idx_v.at[par, pl.ds(j * _CH, _CH)]],
                rows_v.at[par, pl.ds(j * _CH, _CH)], sems[par]).wait()

    fire(0)
    for ch in range(_NCH):
        par = ch % 2
        if ch + 1 < _NCH:
            fire(ch + 1)
        drain(ch)
        base = wid * _PPW + ch * _CH

        def body(p, _):
            wv = wvs[par][pl.ds(3 * p, 16)]
            w0 = wv[0]
            w1 = wv[1]
            w2 = wv[2]
            for c in range(_COUT // 16):
                sl = pl.ds(c * 16, 16)
                y_v[p, sl] = (rows_v[par, p, sl] * w0
                              + rows_v[par, _CH + p, sl] * w1
                              + rows_v[par, 2 * _CH + p, sl] * w2)
            return _

        lax.fori_loop(0, _CH, body, 0)
        pltpu.sync_copy(y_v, y_hbm.at[pl.ds(base, _CH)])


def _mlp_from_y(y_ref, uf_ref, w0_ref, x_ref, sums_ref):
    b = pl.program_id(0)
    nb = pl.program_id(1)

    @pl.when(jnp.logical_and(b == 0, nb == 0))
    def _():
        sums_ref[...] = jnp.zeros_like(sums_ref)

    x = jnp.swapaxes(y_ref[0], 0, 1)
    x = x + jnp.dot(w0_ref[:, _C2:].astype(jnp.bfloat16),
                    uf_ref[0].astype(jnp.bfloat16),
                    preferred_element_type=jnp.float32)
    x_ref[0] = x
    sums_ref[:, 0:1] += jnp.sum(x, axis=1, keepdims=True)
    sums_ref[:, 1:2] += jnp.sum(x * x, axis=1, keepdims=True)


def _fp_norm(x_ref, sums_ref, gm_ref, bt_ref, o_ref):
    cnt = jnp.float32(_B * _N)
    mean = sums_ref[:, 0:1] / cnt
    var = sums_ref[:, 1:2] / cnt - mean * mean
    inv = jax.lax.rsqrt(var + 1e-5)
    scale = gm_ref[...] * inv
    shift = bt_ref[...] - mean * scale
    o_ref[0] = jnp.maximum(x_ref[0] * scale + shift, 0.0)


def kernel(unknown, known, unknow_feats, known_feats, W0, gamma0, beta0):
    uc = jnp.concatenate(
        [jnp.swapaxes(unknown, 1, 2),
         jnp.zeros((_B, 5, _N), jnp.float32)], axis=1)
    kc = jnp.concatenate(
        [known, jnp.zeros((_B, _M, 5), jnp.float32)], axis=2)

    wgt, idx, gt = pl.pallas_call(
        _nn3w,
        grid=(_B, _NB),
        in_specs=[
            pl.BlockSpec((1, _M, 8), lambda b, n: (b, 0, 0)),
            pl.BlockSpec((1, 8, _TN), lambda b, n: (b, 0, n)),
            pl.BlockSpec((1, _C2, _M), lambda b, n: (b, 0, 0)),
            pl.BlockSpec((_COUT, _C1 + _C2), lambda b, n: (0, 0)),
        ],
        out_specs=[
            pl.BlockSpec((1, 8, _TN), lambda b, n: (b, 0, n)),
            pl.BlockSpec((1, 8, _TN), lambda b, n: (b, 0, n)),
            pl.BlockSpec((_M, _COUT), lambda b, n: (b, 0)),
        ],
        out_shape=[
            jax.ShapeDtypeStruct((_B, 8, _N), jnp.float32),
            jax.ShapeDtypeStruct((_B, 8, _N), jnp.int32),
            jax.ShapeDtypeStruct((_B * _M, _COUT), jnp.float32),
        ],
        compiler_params=pltpu.CompilerParams(
            dimension_semantics=("arbitrary", "arbitrary")),
    )(kc, uc, known_feats, W0)

    idx_flat = jnp.swapaxes(idx[:, :3, :], 0, 1).reshape(3 * _B * _N)
    wgt_flat = jnp.swapaxes(wgt[:, :3, :], 1, 2).reshape(_B * _N * 3)

    sc_interp = functools.partial(
        pl.kernel,
        out_type=jax.ShapeDtypeStruct((_B * _N, _COUT), jnp.float32),
        mesh=plsc.VectorSubcoreMesh(core_axis_name="c", subcore_axis_name="s"),
        scratch_types=[
            pltpu.VMEM((2, _CH * 3), jnp.int32),
            pltpu.VMEM((_CH * 3 + 16,), jnp.float32),
            pltpu.VMEM((_CH * 3 + 16,), jnp.float32),
            pltpu.VMEM((2, _CH * 3, _COUT), jnp.float32),
            pltpu.VMEM((_CH, _COUT), jnp.float32),
            pltpu.SemaphoreType.DMA,
            pltpu.SemaphoreType.DMA,
        ],
    )(_interp_sc)

    y = sc_interp(gt, idx_flat, wgt_flat)
    y = y.reshape(_B, _N, _COUT)

    x_pre, sums = pl.pallas_call(
        _mlp_from_y,
        grid=(_B, _NB),
        in_specs=[
            pl.BlockSpec((1, _TN, _COUT), lambda b, n: (b, n, 0)),
            pl.BlockSpec((1, _C1, _TN), lambda b, n: (b, 0, n)),
            pl.BlockSpec((_COUT, _C1 + _C2), lambda b, n: (0, 0)),
        ],
        out_specs=[
            pl.BlockSpec((1, _COUT, _TN), lambda b, n: (b, 0, n)),
            pl.BlockSpec((_COUT, 8), lambda b, n: (0, 0)),
        ],
        out_shape=[
            jax.ShapeDtypeStruct((_B, _COUT, _N), jnp.float32),
            jax.ShapeDtypeStruct((_COUT, 8), jnp.float32),
        ],
        compiler_params=pltpu.CompilerParams(
            dimension_semantics=("arbitrary", "arbitrary")),
    )(y, unknow_feats, W0)

    out = pl.pallas_call(
        _fp_norm,
        grid=(_B, _NB2),
        in_specs=[
            pl.BlockSpec((1, _COUT, _TN2), lambda b, n: (b, 0, n)),
            pl.BlockSpec((_COUT, 8), lambda b, n: (0, 0)),
            pl.BlockSpec((_COUT, 1), lambda b, n: (0, 0)),
            pl.BlockSpec((_COUT, 1), lambda b, n: (0, 0)),
        ],
        out_specs=pl.BlockSpec((1, _COUT, _TN2), lambda b, n: (b, 0, n)),
        out_shape=jax.ShapeDtypeStruct((_B, _COUT, _N), jnp.float32),
        compiler_params=pltpu.CompilerParams(
            dimension_semantics=("arbitrary", "arbitrary")),
    )(x_pre, sums, gamma0.reshape(_COUT, 1), beta0.reshape(_COUT, 1))

    return out


# speedup vs baseline: 1.0857x; 1.0158x over previous
import functools

import jax
import jax.numpy as jnp
from jax import lax
from jax.experimental import pallas as pl
from jax.experimental.pallas import tpu as pltpu
from jax.experimental.pallas import tpu_sc as plsc

_B, _N, _M = 4, 8192, 2048
_C1, _C2 = 64, 128
_COUT = 128
_TN = 512
_NB = _N // _TN
_TN2 = 1024
_NB2 = _N // _TN2

_HI = jax.lax.Precision.HIGHEST

_NW = 32
_PPW = _N // _NW
_CH = 128
_NCH = _PPW // _CH


def _nn3w(kc_ref, uc_ref, kf_ref, w0_ref, w_ref, i_ref, gt_ref):
    nb = pl.program_id(0)

    @pl.when(nb == 0)
    def _():
        kf = kf_ref[...].astype(jnp.bfloat16)
        w0a = w0_ref[:, :_C2].astype(jnp.bfloat16)
        gt_ref[...] = jax.lax.dot_general(
            kf, w0a, (((0,), (1,)), ((), ())),
            preferred_element_type=jnp.float32)

    kc = kc_ref[...]
    uc = uc_ref[...]
    inner = jnp.dot(kc.astype(jnp.bfloat16), uc.astype(jnp.bfloat16),
                    preferred_element_type=jnp.float32)
    k2 = kc[:, 0:1] * kc[:, 0:1] + kc[:, 1:2] * kc[:, 1:2] + kc[:, 2:3] * kc[:, 2:3]
    u2 = uc[0:1, :] * uc[0:1, :] + uc[1:2, :] * uc[1:2, :] + uc[2:3, :] * uc[2:3, :]
    d2 = jnp.maximum(u2 + k2 - 2.0 * inner, 0.0)

    inf = jnp.float32(jnp.inf)
    iota = jax.lax.broadcasted_iota(jnp.int32, (_M, _TN), 0)
    big = jnp.int32(_M)

    m1 = jnp.min(d2, axis=0, keepdims=True)
    i1 = jnp.min(jnp.where(d2 == m1, iota, big), axis=0, keepdims=True)
    d2a = jnp.where(iota == i1, inf, d2)
    m2 = jnp.min(d2a, axis=0, keepdims=True)
    i2 = jnp.min(jnp.where(d2a == m2, iota, big), axis=0, keepdims=True)
    d2b = jnp.where(iota == i2, inf, d2a)
    m3 = jnp.min(d2b, axis=0, keepdims=True)
    i3 = jnp.min(jnp.where(d2b == m3, iota, big), axis=0, keepdims=True)

    r1 = 1.0 / (m1 + 1e-8)
    r2 = 1.0 / (m2 + 1e-8)
    r3 = 1.0 / (m3 + 1e-8)
    norm = r1 + r2 + r3

    w_ref[0:1, :] = r1 / norm
    w_ref[1:2, :] = r2 / norm
    w_ref[2:3, :] = r3 / norm
    w_ref[3:8, :] = jnp.zeros((5, _TN), jnp.float32)
    i_ref[0:1, :] = i1
    i_ref[1:2, :] = i2
    i_ref[2:3, :] = i3
    i_ref[3:8, :] = jnp.zeros((5, _TN), jnp.int32)


def _interp_sc(gt_hbm, idx_hbm, wgt_hbm, y_hbm, idx_v, w_v0, w_v1, rows_v,
               y_v, sem0, sem1):
    wid = lax.axis_index("s") * 2 + lax.axis_index("c")
    sems = (sem0, sem1)
    wvs = (w_v0, w_v1)

    def fire(ch):
        par = ch % 2
        base = wid * _PPW + ch * _CH
        pltpu.sync_copy(wgt_hbm.at[pl.ds(base * 3, _CH * 3)],
                        wvs[par].at[pl.ds(0, _CH * 3)])
        for j in range(3):
            pltpu.sync_copy(idx_hbm.at[pl.ds(j * _N + base, _CH)],
                            idx_v.at[par, pl.ds(j * _CH, _CH)])
            pltpu.async_copy(
                gt_hbm.at[idx_v.at[par, pl.ds(j * _CH, _CH)]],
                rows_v.at[par, pl.ds(j * _CH, _CH)], sems[par])

    def drain(ch):
        par = ch % 2
        for j in range(3):
            pltpu.make_async_copy(
                gt_hbm.at[idx_v.at[par, pl.ds(j * _CH, _CH)]],
                rows_v.at[par, pl.ds(j * _CH, _CH)], sems[par]).wait()

    fire(0)
    for ch in range(_NCH):
        par = ch % 2
        if ch + 1 < _NCH:
            fire(ch + 1)
        drain(ch)
        base = wid * _PPW + ch * _CH

        def body(p, _):
            wv = wvs[par][pl.ds(3 * p, 16)]
            w0 = wv[0]
            w1 = wv[1]
            w2 = wv[2]
            for c in range(_COUT // 16):
                sl = pl.ds(c * 16, 16)
                y_v[p, sl] = (rows_v[par, p, sl] * w0
                              + rows_v[par, _CH + p, sl] * w1
                              + rows_v[par, 2 * _CH + p, sl] * w2)
            return _

        lax.fori_loop(0, _CH, body, 0)
        pltpu.sync_copy(y_v, y_hbm.at[pl.ds(base, _CH)])


def _mlp_from_y(y_ref, uf_ref, w0_ref, x_ref, sums_ref):
    b = pl.program_id(0)
    nb = pl.program_id(1)

    @pl.when(jnp.logical_and(b == 0, nb == 0))
    def _():
        sums_ref[...] = jnp.zeros_like(sums_ref)

    x = jnp.swapaxes(y_ref[0], 0, 1)
    x = x + jnp.dot(w0_ref[:, _C2:].astype(jnp.bfloat16),
                    uf_ref[0].astype(jnp.bfloat16),
                    preferred_element_type=jnp.float32)
    x_ref[0] = x
    sums_ref[:, 0:1] += jnp.sum(x, axis=1, keepdims=True)
    sums_ref[:, 1:2] += jnp.sum(x * x, axis=1, keepdims=True)


def _fp_norm(x_ref, sums_ref, gm_ref, bt_ref, o_ref):
    cnt = jnp.float32(_B * _N)
    mean = sums_ref[:, 0:1] / cnt
    var = sums_ref[:, 1:2] / cnt - mean * mean
    inv = jax.lax.rsqrt(var + 1e-5)
    scale = gm_ref[...] * inv
    shift = bt_ref[...] - mean * scale
    o_ref[0] = jnp.maximum(x_ref[0] * scale + shift, 0.0)


def kernel(unknown, known, unknow_feats, known_feats, W0, gamma0, beta0):
    uc = jnp.concatenate(
        [jnp.swapaxes(unknown, 1, 2),
         jnp.zeros((_B, 5, _N), jnp.float32)], axis=1)
    kc = jnp.concatenate(
        [known, jnp.zeros((_B, _M, 5), jnp.float32)], axis=2)

    nn3w = pl.pallas_call(
        _nn3w,
        grid=(_NB,),
        in_specs=[
            pl.BlockSpec((_M, 8), lambda n: (0, 0)),
            pl.BlockSpec((8, _TN), lambda n: (0, n)),
            pl.BlockSpec((_C2, _M), lambda n: (0, 0)),
            pl.BlockSpec((_COUT, _C1 + _C2), lambda n: (0, 0)),
        ],
        out_specs=[
            pl.BlockSpec((8, _TN), lambda n: (0, n)),
            pl.BlockSpec((8, _TN), lambda n: (0, n)),
            pl.BlockSpec((_M, _COUT), lambda n: (0, 0)),
        ],
        out_shape=[
            jax.ShapeDtypeStruct((8, _N), jnp.float32),
            jax.ShapeDtypeStruct((8, _N), jnp.int32),
            jax.ShapeDtypeStruct((_M, _COUT), jnp.float32),
        ],
        compiler_params=pltpu.CompilerParams(
            dimension_semantics=("arbitrary",)),
    )

    sc_interp = functools.partial(
        pl.kernel,
        out_type=jax.ShapeDtypeStruct((_N, _COUT), jnp.float32),
        mesh=plsc.VectorSubcoreMesh(core_axis_name="c", subcore_axis_name="s"),
        scratch_types=[
            pltpu.VMEM((2, _CH * 3), jnp.int32),
            pltpu.VMEM((_CH * 3 + 16,), jnp.float32),
            pltpu.VMEM((_CH * 3 + 16,), jnp.float32),
            pltpu.VMEM((2, _CH * 3, _COUT), jnp.float32),
            pltpu.VMEM((_CH, _COUT), jnp.float32),
            pltpu.SemaphoreType.DMA,
            pltpu.SemaphoreType.DMA,
        ],
    )(_interp_sc)

    ys = []
    for b in range(_B):
        wgt, idx, gt = nn3w(kc[b], uc[b], known_feats[b], W0)
        idx_flat = idx[:3, :].reshape(3 * _N)
        wgt_flat = jnp.swapaxes(wgt[:3, :], 0, 1).reshape(_N * 3)
        ys.append(sc_interp(gt, idx_flat, wgt_flat))

    y = jnp.stack(ys, axis=0)

    x_pre, sums = pl.pallas_call(
        _mlp_from_y,
        grid=(_B, _NB),
        in_specs=[
            pl.BlockSpec((1, _TN, _COUT), lambda b, n: (b, n, 0)),
            pl.BlockSpec((1, _C1, _TN), lambda b, n: (b, 0, n)),
            pl.BlockSpec((_COUT, _C1 + _C2), lambda b, n: (0, 0)),
        ],
        out_specs=[
            pl.BlockSpec((1, _COUT, _TN), lambda b, n: (b, 0, n)),
            pl.BlockSpec((_COUT, 8), lambda b, n: (0, 0)),
        ],
        out_shape=[
            jax.ShapeDtypeStruct((_B, _COUT, _N), jnp.float32),
            jax.ShapeDtypeStruct((_COUT, 8), jnp.float32),
        ],
        compiler_params=pltpu.CompilerParams(
            dimension_semantics=("arbitrary", "arbitrary")),
    )(y, unknow_feats, W0)

    out = pl.pallas_call(
        _fp_norm,
        grid=(_B, _NB2),
        in_specs=[
            pl.BlockSpec((1, _COUT, _TN2), lambda b, n: (b, 0, n)),
            pl.BlockSpec((_COUT, 8), lambda b, n: (0, 0)),
            pl.BlockSpec((_COUT, 1), lambda b, n: (0, 0)),
            pl.BlockSpec((_COUT, 1), lambda b, n: (0, 0)),
        ],
        out_specs=pl.BlockSpec((1, _COUT, _TN2), lambda b, n: (b, 0, n)),
        out_shape=jax.ShapeDtypeStruct((_B, _COUT, _N), jnp.float32),
        compiler_params=pltpu.CompilerParams(
            dimension_semantics=("arbitrary", "arbitrary")),
    )(x_pre, sums, gamma0.reshape(_COUT, 1), beta0.reshape(_COUT, 1))

    return out


# speedup vs baseline: 1.2824x; 1.1812x over previous
import functools

import jax
import jax.numpy as jnp
from jax import lax
from jax.experimental import pallas as pl
from jax.experimental.pallas import tpu as pltpu
from jax.experimental.pallas import tpu_sc as plsc

_B, _N, _M = 4, 8192, 2048
_C1, _C2 = 64, 128
_COUT = 128
_TN = 512
_NB = _N // _TN
_TN2 = 1024
_NB2 = _N // _TN2

_HI = jax.lax.Precision.HIGHEST

_NW = 32
_PPW = _N // _NW
_CH = 128
_NCH = _PPW // _CH


def _nn3w(kc_ref, uc_ref, kf_ref, w0_ref, w_ref, i_ref, gt_ref):
    nb = pl.program_id(0)

    @pl.when(nb == 0)
    def _():
        kf = kf_ref[...].astype(jnp.bfloat16)
        w0a = w0_ref[:, :_C2].astype(jnp.bfloat16)
        gt_ref[...] = jax.lax.dot_general(
            kf, w0a, (((0,), (1,)), ((), ())),
            preferred_element_type=jnp.float32)

    kc = kc_ref[...]
    uc = uc_ref[...]
    inner = jnp.dot(kc.astype(jnp.bfloat16), uc.astype(jnp.bfloat16),
                    preferred_element_type=jnp.float32)
    k2 = kc[:, 0:1] * kc[:, 0:1] + kc[:, 1:2] * kc[:, 1:2] + kc[:, 2:3] * kc[:, 2:3]
    u2 = uc[0:1, :] * uc[0:1, :] + uc[1:2, :] * uc[1:2, :] + uc[2:3, :] * uc[2:3, :]
    d2 = jnp.maximum(u2 + k2 - 2.0 * inner, 0.0)

    iota = jax.lax.broadcasted_iota(jnp.int32, (_M, _TN), 0)

    kbits = jax.lax.bitcast_convert_type(d2, jnp.int32)
    ikey = (kbits & jnp.int32(~2047)) | iota
    maxk = jnp.int32(0x7FFFFFFF)
    k1 = jnp.min(ikey, axis=0, keepdims=True)
    ikey2 = jnp.where(ikey == k1, maxk, ikey)
    k2 = jnp.min(ikey2, axis=0, keepdims=True)
    ikey3 = jnp.where(ikey2 == k2, maxk, ikey2)
    k3 = jnp.min(ikey3, axis=0, keepdims=True)

    lowm = jnp.int32(2047)
    i1 = k1 & lowm
    i2 = k2 & lowm
    i3 = k3 & lowm
    m1 = jax.lax.bitcast_convert_type(k1 & ~lowm, jnp.float32)
    m2 = jax.lax.bitcast_convert_type(k2 & ~lowm, jnp.float32)
    m3 = jax.lax.bitcast_convert_type(k3 & ~lowm, jnp.float32)

    r1 = 1.0 / (m1 + 1e-8)
    r2 = 1.0 / (m2 + 1e-8)
    r3 = 1.0 / (m3 + 1e-8)
    norm = r1 + r2 + r3

    w_ref[0:1, :] = r1 / norm
    w_ref[1:2, :] = r2 / norm
    w_ref[2:3, :] = r3 / norm
    w_ref[3:8, :] = jnp.zeros((5, _TN), jnp.float32)
    i_ref[0:1, :] = i1
    i_ref[1:2, :] = i2
    i_ref[2:3, :] = i3
    i_ref[3:8, :] = jnp.zeros((5, _TN), jnp.int32)


def _interp_sc(gt_hbm, idx_hbm, wgt_hbm, y_hbm, idx_v, w_v0, w_v1, rows_v,
               y_v, sem0, sem1):
    wid = lax.axis_index("s") * 2 + lax.axis_index("c")
    sems = (sem0, sem1)
    wvs = (w_v0, w_v1)

    def fire(ch):
        par = ch % 2
        base = wid * _PPW + ch * _CH
        pltpu.sync_copy(wgt_hbm.at[pl.ds(base * 3, _CH * 3)],
                        wvs[par].at[pl.ds(0, _CH * 3)])
        for j in range(3):
            pltpu.sync_copy(idx_hbm.at[pl.ds(j * _N + base, _CH)],
                            idx_v.at[par, pl.ds(j * _CH, _CH)])
            pltpu.async_copy(
                gt_hbm.at[idx_v.at[par, pl.ds(j * _CH, _CH)]],
                rows_v.at[par, pl.ds(j * _CH, _CH)], sems[par])

    def drain(ch):
        par = ch % 2
        for j in range(3):
            pltpu.make_async_copy(
                gt_hbm.at[idx_v.at[par, pl.ds(j * _CH, _CH)]],
                rows_v.at[par, pl.ds(j * _CH, _CH)], sems[par]).wait()

    fire(0)
    for ch in range(_NCH):
        par = ch % 2
        if ch + 1 < _NCH:
            fire(ch + 1)
        drain(ch)
        base = wid * _PPW + ch * _CH

        def body(p, _):
            wv = wvs[par][pl.ds(3 * p, 16)]
            w0 = wv[0]
            w1 = wv[1]
            w2 = wv[2]
            for c in range(_COUT // 16):
                sl = pl.ds(c * 16, 16)
                y_v[p, sl] = (rows_v[par, p, sl] * w0
                              + rows_v[par, _CH + p, sl] * w1
                              + rows_v[par, 2 * _CH + p, sl] * w2)
            return _

        lax.fori_loop(0, _CH, body, 0)
        pltpu.sync_copy(y_v, y_hbm.at[pl.ds(base, _CH)])


def _mlp_from_y(y_ref, uf_ref, w0_ref, x_ref, sums_ref):
    b = pl.program_id(0)
    nb = pl.program_id(1)

    @pl.when(jnp.logical_and(b == 0, nb == 0))
    def _():
        sums_ref[...] = jnp.zeros_like(sums_ref)

    x = jnp.swapaxes(y_ref[0], 0, 1)
    x = x + jnp.dot(w0_ref[:, _C2:].astype(jnp.bfloat16),
                    uf_ref[0].astype(jnp.bfloat16),
                    preferred_element_type=jnp.float32)
    x_ref[0] = x
    sums_ref[:, 0:1] += jnp.sum(x, axis=1, keepdims=True)
    sums_ref[:, 1:2] += jnp.sum(x * x, axis=1, keepdims=True)


def _fp_norm(x_ref, sums_ref, gm_ref, bt_ref, o_ref):
    cnt = jnp.float32(_B * _N)
    mean = sums_ref[:, 0:1] / cnt
    var = sums_ref[:, 1:2] / cnt - mean * mean
    inv = jax.lax.rsqrt(var + 1e-5)
    scale = gm_ref[...] * inv
    shift = bt_ref[...] - mean * scale
    o_ref[0] = jnp.maximum(x_ref[0] * scale + shift, 0.0)


def kernel(unknown, known, unknow_feats, known_feats, W0, gamma0, beta0):
    uc = jnp.concatenate(
        [jnp.swapaxes(unknown, 1, 2),
         jnp.zeros((_B, 5, _N), jnp.float32)], axis=1)
    kc = jnp.concatenate(
        [known, jnp.zeros((_B, _M, 5), jnp.float32)], axis=2)

    nn3w = pl.pallas_call(
        _nn3w,
        grid=(_NB,),
        in_specs=[
            pl.BlockSpec((_M, 8), lambda n: (0, 0)),
            pl.BlockSpec((8, _TN), lambda n: (0, n)),
            pl.BlockSpec((_C2, _M), lambda n: (0, 0)),
            pl.BlockSpec((_COUT, _C1 + _C2), lambda n: (0, 0)),
        ],
        out_specs=[
            pl.BlockSpec((8, _TN), lambda n: (0, n)),
            pl.BlockSpec((8, _TN), lambda n: (0, n)),
            pl.BlockSpec((_M, _COUT), lambda n: (0, 0)),
        ],
        out_shape=[
            jax.ShapeDtypeStruct((8, _N), jnp.float32),
            jax.ShapeDtypeStruct((8, _N), jnp.int32),
            jax.ShapeDtypeStruct((_M, _COUT), jnp.float32),
        ],
        compiler_params=pltpu.CompilerParams(
            dimension_semantics=("arbitrary",)),
    )

    sc_interp = functools.partial(
        pl.kernel,
        out_type=jax.ShapeDtypeStruct((_N, _COUT), jnp.float32),
        mesh=plsc.VectorSubcoreMesh(core_axis_name="c", subcore_axis_name="s"),
        scratch_types=[
            pltpu.VMEM((2, _CH * 3), jnp.int32),
            pltpu.VMEM((_CH * 3 + 16,), jnp.float32),
            pltpu.VMEM((_CH * 3 + 16,), jnp.float32),
            pltpu.VMEM((2, _CH * 3, _COUT), jnp.float32),
            pltpu.VMEM((_CH, _COUT), jnp.float32),
            pltpu.SemaphoreType.DMA,
            pltpu.SemaphoreType.DMA,
        ],
    )(_interp_sc)

    ys = []
    for b in range(_B):
        wgt, idx, gt = nn3w(kc[b], uc[b], known_feats[b], W0)
        idx_flat = idx[:3, :].reshape(3 * _N)
        wgt_flat = jnp.swapaxes(wgt[:3, :], 0, 1).reshape(_N * 3)
        ys.append(sc_interp(gt, idx_flat, wgt_flat))

    y = jnp.stack(ys, axis=0)

    x_pre, sums = pl.pallas_call(
        _mlp_from_y,
        grid=(_B, _NB),
        in_specs=[
            pl.BlockSpec((1, _TN, _COUT), lambda b, n: (b, n, 0)),
            pl.BlockSpec((1, _C1, _TN), lambda b, n: (b, 0, n)),
            pl.BlockSpec((_COUT, _C1 + _C2), lambda b, n: (0, 0)),
        ],
        out_specs=[
            pl.BlockSpec((1, _COUT, _TN), lambda b, n: (b, 0, n)),
            pl.BlockSpec((_COUT, 8), lambda b, n: (0, 0)),
        ],
        out_shape=[
            jax.ShapeDtypeStruct((_B, _COUT, _N), jnp.float32),
            jax.ShapeDtypeStruct((_COUT, 8), jnp.float32),
        ],
        compiler_params=pltpu.CompilerParams(
            dimension_semantics=("arbitrary", "arbitrary")),
    )(y, unknow_feats, W0)

    out = pl.pallas_call(
        _fp_norm,
        grid=(_B, _NB2),
        in_specs=[
            pl.BlockSpec((1, _COUT, _TN2), lambda b, n: (b, 0, n)),
            pl.BlockSpec((_COUT, 8), lambda b, n: (0, 0)),
            pl.BlockSpec((_COUT, 1), lambda b, n: (0, 0)),
            pl.BlockSpec((_COUT, 1), lambda b, n: (0, 0)),
        ],
        out_specs=pl.BlockSpec((1, _COUT, _TN2), lambda b, n: (b, 0, n)),
        out_shape=jax.ShapeDtypeStruct((_B, _COUT, _N), jnp.float32),
        compiler_params=pltpu.CompilerParams(
            dimension_semantics=("arbitrary", "arbitrary")),
    )(x_pre, sums, gamma0.reshape(_COUT, 1), beta0.reshape(_COUT, 1))

    return out
